# baseline (speedup 1.0000x reference)
import jax
import jax.numpy as jnp
from jax import lax
from jax.experimental import pallas as pl
from jax.experimental.pallas import tpu as pltpu, tpu_sc as plsc

N = 10000
E = 160000
F = 128
C = 4

NC = 2
NS = 16
NW = NC * NS
L = 16

R = 160
NCHUNK = (N + R - 1) // R
LAST_ROWS = N - (NCHUNK - 1) * R

B = 1000
NB = E // B
G = 128
MATCAP = B + G + L


def _matmul_body(x_ref, w_ref, o_ref):
    o_ref[...] = jnp.dot(x_ref[...], w_ref[...].T,
                         preferred_element_type=jnp.float32)


def _linear(x, W_lin):
    return pl.pallas_call(
        _matmul_body,
        out_shape=jax.ShapeDtypeStruct((N, F), jnp.float32),
        grid=(10,),
        in_specs=[
            pl.BlockSpec((N // 10, F), lambda i: (i, 0)),
            pl.BlockSpec((F, F), lambda i: (0, 0)),
        ],
        out_specs=pl.BlockSpec((N // 10, F), lambda i: (i, 0)),
    )(x, W_lin)


SHIFT = 14
CMASK = (1 << SHIFT) - 1


def _sc_body(h_hbm, pk_hbm, watt_hbm, out_hbm,
             acc, h_chunk, ebuf0, ebuf1, mat_pk, gidx,
             pend_pk, hc_buf, wv, esem0, esem1, gsem):
    wid = lax.axis_index("s") * NC + lax.axis_index("c")
    pltpu.sync_copy(watt_hbm, wv)
    wvs = [[wv[c, pl.ds(k * L, L)] for k in range(F // L)] for c in range(C)]
    esems = (esem0, esem1)
    ebufs = (ebuf0, ebuf1)

    def compute_batch(lr_ref, p, cnt):
        @plsc.parallel_loop(0, cnt, unroll=1)
        def edge_body(e):
            lr = lr_ref[pl.ds(p + e, L)][0] >> SHIFT
            hr = [h_chunk[lr, pl.ds(k * L, L)] for k in range(F // L)]
            hc = [hc_buf[e, pl.ds(k * L, L)] for k in range(F // L)]
            t = [jnp.maximum(a + b, 0.0) for a, b in zip(hr, hc)]
            sv = []
            for c in range(C):
                s = t[0] * wvs[c][0]
                for k in range(1, F // L):
                    s = s + t[k] * wvs[c][k]
                sv.append(jnp.full((L,), jnp.sum(s)))
            m = jnp.maximum(jnp.maximum(sv[0], sv[1]),
                            jnp.maximum(sv[2], sv[3]))
            ev = [jnp.exp(s - m) for s in sv]
            inv = 1.0 / (ev[0] + ev[1] + ev[2] + ev[3])
            for c in range(C):
                ac = ev[c] * inv
                for k in range(F // L):
                    plsc.addupdate(acc.at[lr, pl.ds(c * F + k * L, L)],
                                   ac * hc[k])

    def fill_gidx(p):
        for k in range(G // L):
            gidx[pl.ds(k * L, L)] = mat_pk[pl.ds(p + k * L, L)] & CMASK

    def wait_gather():
        pltpu.make_async_copy(h_hbm.at[gidx], hc_buf, gsem).wait()

    def do_chunk(chunk):
        row0 = chunk * R

        @plsc.parallel_loop(0, R, unroll=1)
        def zrow(i):
            for k in range(C * F // L):
                acc[i, pl.ds(k * L, L)] = jnp.zeros((L,), jnp.float32)

        @pl.when(chunk < NCHUNK - 1)
        def _():
            pltpu.sync_copy(h_hbm.at[pl.ds(row0, R)], h_chunk)
        @pl.when(chunk == NCHUNK - 1)
        def _():
            pltpu.sync_copy(h_hbm.at[pl.ds(row0, LAST_ROWS)],
                            h_chunk.at[pl.ds(0, LAST_ROWS)])

        @plsc.parallel_loop(0, R, unroll=1)
        def hscale(i):
            for k in range(F // L):
                h_chunk[i, pl.ds(k * L, L)] = 0.5 * h_chunk[i, pl.ds(k * L, L)]

        plo = row0 << SHIFT
        phi = (row0 + R) << SHIFT

        def issue_block(b, par):
            pltpu.async_copy(pk_hbm.at[pl.ds(b * B, B)], ebufs[par],
                             esems[par])

        def wait_block(par):
            pltpu.make_async_copy(pk_hbm.at[pl.ds(0, B)], ebufs[par],
                                  esems[par]).wait()

        def handle_block(b, par, carry):
            off, pend = carry

            @pl.when(b + 1 < NB)
            def _():
                issue_block(b + 1, 1 - par)
            wait_block(par)

            @plsc.parallel_loop(0, B // L, unroll=2, carry=off)
            def filt(j, off):
                pv = ebufs[par][pl.ds(j * L, L)]
                mask = (pv >= plo) & (pv < phi)
                cs = plsc.cumsum(mask.astype(jnp.int32))
                pos = off + cs - 1
                plsc.store_scatter(mat_pk, [pos], pv - plo, mask=mask)
                return off + cs[L - 1]

            off = filt

            @pl.when(pend > 0)
            def _():
                wait_gather()
                compute_batch(pend_pk, 0, G)

            nextra = jnp.maximum(off // G - 1, 0)

            def extra(i, off):
                p = off - G
                fill_gidx(p)
                pltpu.async_copy(h_hbm.at[gidx], hc_buf, gsem).wait()
                compute_batch(mat_pk, p, G)
                return off - G

            off = lax.fori_loop(0, nextra, extra, off)

            have = (off >= G).astype(jnp.int32)

            @pl.when(off >= G)
            def _():
                p = off - G
                fill_gidx(p)
                for k in range(G // L):
                    pend_pk[pl.ds(k * L, L)] = mat_pk[pl.ds(p + k * L, L)]
                pltpu.async_copy(h_hbm.at[gidx], hc_buf, gsem)

            off = jnp.where(off >= G, off - G, off)
            return (off, have)

        issue_block(0, 0)

        def block_pair(i, carry):
            carry = handle_block(2 * i, 0, carry)
            carry = handle_block(2 * i + 1, 1, carry)
            return carry

        off, pend = lax.fori_loop(0, NB // 2, block_pair,
                                  (jnp.int32(0), jnp.int32(0)))

        @pl.when(pend > 0)
        def _():
            wait_gather()
            compute_batch(pend_pk, 0, G)

        lane = lax.iota(jnp.int32, L)

        @pl.when(off > 0)
        def _():
            for k in range(G // L):
                cvv = mat_pk[pl.ds(k * L, L)] & CMASK
                gl = lane + k * L
                gidx[pl.ds(k * L, L)] = jnp.where(gl < off, cvv, gl)
            pltpu.async_copy(h_hbm.at[gidx], hc_buf, gsem).wait()
            compute_batch(mat_pk, 0, off)

        @pl.when(chunk < NCHUNK - 1)
        def _():
            pltpu.sync_copy(acc, out_hbm.at[pl.ds(row0, R)])
        @pl.when(chunk == NCHUNK - 1)
        def _():
            pltpu.sync_copy(acc.at[pl.ds(0, LAST_ROWS)],
                            out_hbm.at[pl.ds(row0, LAST_ROWS)])

    for slot in range(2):
        chunk = wid + NW * slot
        @pl.when(chunk < NCHUNK)
        def _():
            do_chunk(chunk)


@jax.jit
def _sc_aggregate(h, pk, W_att):
    mesh = plsc.VectorSubcoreMesh(core_axis_name="c", subcore_axis_name="s")
    f = pl.kernel(
        _sc_body,
        out_type=jax.ShapeDtypeStruct((N, C * F), jnp.float32),
        mesh=mesh,
        compiler_params=pltpu.CompilerParams(needs_layout_passes=False),
        scratch_types=[
            pltpu.VMEM((R, C * F), jnp.float32),
            pltpu.VMEM((R, F), jnp.float32),
            pltpu.VMEM((B,), jnp.int32),
            pltpu.VMEM((B,), jnp.int32),
            pltpu.VMEM((MATCAP,), jnp.int32),
            pltpu.VMEM((G,), jnp.int32),
            pltpu.VMEM((G,), jnp.int32),
            pltpu.VMEM((G, F), jnp.float32),
            pltpu.VMEM((C, F), jnp.float32),
            pltpu.SemaphoreType.DMA,
            pltpu.SemaphoreType.DMA,
            pltpu.SemaphoreType.DMA,
        ],
    )
    return f(h, pk, W_att)


def kernel(x, edge_index, W_lin, W_att):
    h = _linear(x, W_lin)
    pk = (edge_index[0] << SHIFT) | edge_index[1]
    return _sc_aggregate(h, pk, W_att)


# speedup vs baseline: 1.0790x; 1.0790x over previous
import jax
import jax.numpy as jnp
from jax import lax
from jax.experimental import pallas as pl
from jax.experimental.pallas import tpu as pltpu, tpu_sc as plsc

N = 10000
E = 160000
F = 128
C = 4

NC = 2
NS = 16
NW = NC * NS
L = 16

R = 160
NCHUNK = (N + R - 1) // R
LAST_ROWS = N - (NCHUNK - 1) * R

B = 1600
NB = E // B
G = 128
MATCAP = B + G + L


def _matmul_body(x_ref, w_ref, o_ref):
    o_ref[...] = jnp.dot(x_ref[...], w_ref[...].T,
                         preferred_element_type=jnp.float32)


def _linear(x, W_lin):
    return pl.pallas_call(
        _matmul_body,
        out_shape=jax.ShapeDtypeStruct((N, F), jnp.float32),
        grid=(10,),
        in_specs=[
            pl.BlockSpec((N // 10, F), lambda i: (i, 0)),
            pl.BlockSpec((F, F), lambda i: (0, 0)),
        ],
        out_specs=pl.BlockSpec((N // 10, F), lambda i: (i, 0)),
    )(x, W_lin)


SHIFT = 14
CMASK = (1 << SHIFT) - 1


def _sc_body(h_hbm, pk_hbm, watt_hbm, out_hbm,
             acc, h_chunk, ebuf0, ebuf1, mat_pk, gidx,
             pend_pk, hc_buf, wv, esem0, esem1, gsem):
    wid = lax.axis_index("s") * NC + lax.axis_index("c")
    pltpu.sync_copy(watt_hbm, wv)
    wvs = [[wv[c, pl.ds(k * L, L)] for k in range(F // L)] for c in range(C)]
    esems = (esem0, esem1)
    ebufs = (ebuf0, ebuf1)

    def compute_batch(lr_ref, p, cnt):
        @plsc.parallel_loop(0, cnt, unroll=1)
        def edge_body(e):
            lr = lr_ref[pl.ds(p + e, L)][0] >> SHIFT
            hr = [h_chunk[lr, pl.ds(k * L, L)] for k in range(F // L)]
            hc = [hc_buf[e, pl.ds(k * L, L)] for k in range(F // L)]
            t = [jnp.maximum(a + b, 0.0) for a, b in zip(hr, hc)]
            sv = []
            for c in range(C):
                s = t[0] * wvs[c][0]
                for k in range(1, F // L):
                    s = s + t[k] * wvs[c][k]
                sv.append(jnp.full((L,), jnp.sum(s)))
            m = jnp.maximum(jnp.maximum(sv[0], sv[1]),
                            jnp.maximum(sv[2], sv[3]))
            ev = [jnp.exp(s - m) for s in sv]
            inv = 1.0 / (ev[0] + ev[1] + ev[2] + ev[3])
            for c in range(C):
                ac = ev[c] * inv
                for k in range(F // L):
                    plsc.addupdate(acc.at[lr, pl.ds(c * F + k * L, L)],
                                   ac * hc[k])

    def fill_gidx(p):
        for k in range(G // L):
            gidx[pl.ds(k * L, L)] = mat_pk[pl.ds(p + k * L, L)] & CMASK

    def wait_gather():
        pltpu.make_async_copy(h_hbm.at[gidx], hc_buf, gsem).wait()

    def do_chunk(chunk):
        row0 = chunk * R

        @plsc.parallel_loop(0, R, unroll=1)
        def zrow(i):
            for k in range(C * F // L):
                acc[i, pl.ds(k * L, L)] = jnp.zeros((L,), jnp.float32)

        @pl.when(chunk < NCHUNK - 1)
        def _():
            pltpu.sync_copy(h_hbm.at[pl.ds(row0, R)], h_chunk)
        @pl.when(chunk == NCHUNK - 1)
        def _():
            pltpu.sync_copy(h_hbm.at[pl.ds(row0, LAST_ROWS)],
                            h_chunk.at[pl.ds(0, LAST_ROWS)])

        @plsc.parallel_loop(0, R, unroll=1)
        def hscale(i):
            for k in range(F // L):
                h_chunk[i, pl.ds(k * L, L)] = 0.5 * h_chunk[i, pl.ds(k * L, L)]

        plo = row0 << SHIFT
        phi = (row0 + R) << SHIFT

        def issue_block(b, par):
            pltpu.async_copy(pk_hbm.at[pl.ds(b * B, B)], ebufs[par],
                             esems[par])

        def wait_block(par):
            pltpu.make_async_copy(pk_hbm.at[pl.ds(0, B)], ebufs[par],
                                  esems[par]).wait()

        def handle_block(b, par, carry):
            off, pend = carry

            @pl.when(b + 1 < NB)
            def _():
                issue_block(b + 1, 1 - par)
            wait_block(par)

            @plsc.parallel_loop(0, B // L, unroll=2, carry=off)
            def filt(j, off):
                pv = ebufs[par][pl.ds(j * L, L)]
                mask = (pv >= plo) & (pv < phi)
                cs = plsc.cumsum(mask.astype(jnp.int32))
                pos = off + cs - 1
                plsc.store_scatter(mat_pk, [pos], pv - plo, mask=mask)
                return off + cs[L - 1]

            off = filt

            @pl.when(pend > 0)
            def _():
                wait_gather()
                compute_batch(pend_pk, 0, G)

            nextra = jnp.maximum(off // G - 1, 0)

            def extra(i, off):
                p = off - G
                fill_gidx(p)
                pltpu.async_copy(h_hbm.at[gidx], hc_buf, gsem).wait()
                compute_batch(mat_pk, p, G)
                return off - G

            off = lax.fori_loop(0, nextra, extra, off)

            have = (off >= G).astype(jnp.int32)

            @pl.when(off >= G)
            def _():
                p = off - G
                fill_gidx(p)
                for k in range(G // L):
                    pend_pk[pl.ds(k * L, L)] = mat_pk[pl.ds(p + k * L, L)]
                pltpu.async_copy(h_hbm.at[gidx], hc_buf, gsem)

            off = jnp.where(off >= G, off - G, off)
            return (off, have)

        issue_block(0, 0)

        def block_pair(i, carry):
            carry = handle_block(2 * i, 0, carry)
            carry = handle_block(2 * i + 1, 1, carry)
            return carry

        off, pend = lax.fori_loop(0, NB // 2, block_pair,
                                  (jnp.int32(0), jnp.int32(0)))

        @pl.when(pend > 0)
        def _():
            wait_gather()
            compute_batch(pend_pk, 0, G)

        lane = lax.iota(jnp.int32, L)

        @pl.when(off > 0)
        def _():
            for k in range(G // L):
                cvv = mat_pk[pl.ds(k * L, L)] & CMASK
                gl = lane + k * L
                gidx[pl.ds(k * L, L)] = jnp.where(gl < off, cvv, gl)
            pltpu.async_copy(h_hbm.at[gidx], hc_buf, gsem).wait()
            compute_batch(mat_pk, 0, off)

        @pl.when(chunk < NCHUNK - 1)
        def _():
            pltpu.sync_copy(acc, out_hbm.at[pl.ds(row0, R)])
        @pl.when(chunk == NCHUNK - 1)
        def _():
            pltpu.sync_copy(acc.at[pl.ds(0, LAST_ROWS)],
                            out_hbm.at[pl.ds(row0, LAST_ROWS)])

    for slot in range(2):
        chunk = wid + NW * slot
        @pl.when(chunk < NCHUNK)
        def _():
            do_chunk(chunk)


@jax.jit
def _sc_aggregate(h, pk, W_att):
    mesh = plsc.VectorSubcoreMesh(core_axis_name="c", subcore_axis_name="s")
    f = pl.kernel(
        _sc_body,
        out_type=jax.ShapeDtypeStruct((N, C * F), jnp.float32),
        mesh=mesh,
        compiler_params=pltpu.CompilerParams(needs_layout_passes=False),
        scratch_types=[
            pltpu.VMEM((R, C * F), jnp.float32),
            pltpu.VMEM((R, F), jnp.float32),
            pltpu.VMEM((B,), jnp.int32),
            pltpu.VMEM((B,), jnp.int32),
            pltpu.VMEM((MATCAP,), jnp.int32),
            pltpu.VMEM((G,), jnp.int32),
            pltpu.VMEM((G,), jnp.int32),
            pltpu.VMEM((G, F), jnp.float32),
            pltpu.VMEM((C, F), jnp.float32),
            pltpu.SemaphoreType.DMA,
            pltpu.SemaphoreType.DMA,
            pltpu.SemaphoreType.DMA,
        ],
    )
    return f(h, pk, W_att)


def kernel(x, edge_index, W_lin, W_att):
    h = _linear(x, W_lin)
    pk = (edge_index[0] << SHIFT) | edge_index[1]
    return _sc_aggregate(h, pk, W_att)
